# baseline (speedup 1.0000x reference)
import dataclasses
import functools

import jax
import jax.numpy as jnp
from jax import lax
from jax.experimental import pallas as pl
from jax.experimental.pallas import tpu as pltpu
from jax.experimental.pallas import tpu_sc as plsc

NC = 2
NS = 16
NW = NC * NS
L = 16
W = 32


def kernel(h, r, t, ent_re, ent_im, rel_re, rel_im):
    B = h.shape[0]
    D = ent_re.shape[1]
    DP = 2 * D
    assert B % NW == 0
    b_per_w = B // NW
    assert b_per_w % W == 0
    nwin = b_per_w // W
    nchunk = D // L

    ere2 = ent_re.reshape(-1, DP)
    eim2 = ent_im.reshape(-1, DP)
    rre2 = rel_re.reshape(-1, DP)
    rim2 = rel_im.reshape(-1, DP)

    h32 = h.astype(jnp.int32)
    r32 = r.astype(jnp.int32)
    t32 = t.astype(jnp.int32)
    hq, hp = h32 >> 1, h32 & 1
    rq, rp = r32 >> 1, r32 & 1
    tq, tp = t32 >> 1, t32 & 1

    mesh = plsc.VectorSubcoreMesh(core_axis_name="c", subcore_axis_name="s")
    cp = pltpu.CompilerParams()
    if "needs_layout_passes" in pltpu.CompilerParams.__dataclass_fields__:
        cp = dataclasses.replace(cp, needs_layout_passes=False)
    if "use_tc_tiling_on_sc" in pltpu.CompilerParams.__dataclass_fields__:
        cp = dataclasses.replace(cp, use_tc_tiling_on_sc=False)

    @functools.partial(
        pl.kernel,
        mesh=mesh,
        compiler_params=cp,
        out_type=jax.ShapeDtypeStruct((B,), jnp.float32),
        scratch_types=[
            pltpu.VMEM((b_per_w,), jnp.int32),
            pltpu.VMEM((b_per_w,), jnp.int32),
            pltpu.VMEM((b_per_w,), jnp.int32),
            pltpu.VMEM((b_per_w,), jnp.int32),
            pltpu.VMEM((b_per_w,), jnp.int32),
            pltpu.VMEM((b_per_w,), jnp.int32),
            pltpu.VMEM((2, W, DP), jnp.float32),
            pltpu.VMEM((2, W, DP), jnp.float32),
            pltpu.VMEM((2, W, DP), jnp.float32),
            pltpu.VMEM((2, W, DP), jnp.float32),
            pltpu.VMEM((2, W, DP), jnp.float32),
            pltpu.VMEM((2, W, DP), jnp.float32),
            pltpu.VMEM((b_per_w,), jnp.float32),
            pltpu.SemaphoreType.DMA((2,)),
        ],
    )
    def kge_kernel(hq_hbm, hp_hbm, rq_hbm, rp_hbm, tq_hbm, tp_hbm,
                   ere_hbm, eim_hbm, rre_hbm, rim_hbm, out_hbm,
                   hqv, hpv, rqv, rpv, tqv, tpv,
                   bhr, bhi, brr, bri, btr, bti,
                   outv, sems):
        wid = lax.axis_index("s") * NC + lax.axis_index("c")
        base = wid * b_per_w
        pltpu.sync_copy(hq_hbm.at[pl.ds(base, b_per_w)], hqv)
        pltpu.sync_copy(hp_hbm.at[pl.ds(base, b_per_w)], hpv)
        pltpu.sync_copy(rq_hbm.at[pl.ds(base, b_per_w)], rqv)
        pltpu.sync_copy(rp_hbm.at[pl.ds(base, b_per_w)], rpv)
        pltpu.sync_copy(tq_hbm.at[pl.ds(base, b_per_w)], tqv)
        pltpu.sync_copy(tp_hbm.at[pl.ds(base, b_per_w)], tpv)

        def start_gathers(g, slot):
            hs = hqv.at[pl.ds(g * W, W)]
            rs = rqv.at[pl.ds(g * W, W)]
            ts = tqv.at[pl.ds(g * W, W)]
            sem = sems.at[slot]
            pltpu.async_copy(ere_hbm.at[hs], bhr.at[slot], sem)
            pltpu.async_copy(eim_hbm.at[hs], bhi.at[slot], sem)
            pltpu.async_copy(rre_hbm.at[rs], brr.at[slot], sem)
            pltpu.async_copy(rim_hbm.at[rs], bri.at[slot], sem)
            pltpu.async_copy(ere_hbm.at[ts], btr.at[slot], sem)
            pltpu.async_copy(eim_hbm.at[ts], bti.at[slot], sem)

        def drain(slot):
            hs = hqv.at[pl.ds(0, W)]
            rs = rqv.at[pl.ds(0, W)]
            pltpu.make_async_copy(ere_hbm.at[hs], bhr.at[slot], sems.at[slot]).wait()
            pltpu.make_async_copy(eim_hbm.at[hs], bhi.at[slot], sems.at[slot]).wait()
            pltpu.make_async_copy(rre_hbm.at[rs], brr.at[slot], sems.at[slot]).wait()
            pltpu.make_async_copy(rim_hbm.at[rs], bri.at[slot], sems.at[slot]).wait()
            pltpu.make_async_copy(ere_hbm.at[hs], btr.at[slot], sems.at[slot]).wait()
            pltpu.make_async_copy(eim_hbm.at[hs], bti.at[slot], sems.at[slot]).wait()

        lane = lax.iota(jnp.int32, L)
        m0 = lane == 0

        def compute(g, slot):
            sv = jnp.full((L,), slot, jnp.int32)

            @pl.loop(0, W)
            def _(w):
                pos = g * W + w
                posv = jnp.broadcast_to(pos, (L,))
                wv = jnp.broadcast_to(w, (L,))
                bh = plsc.load_gather(hpv, [posv]) * D + lane
                br = plsc.load_gather(rpv, [posv]) * D + lane
                bt = plsc.load_gather(tpv, [posv]) * D + lane
                acc = jnp.zeros((L,), jnp.float32)
                for c in range(nchunk):
                    off = c * L
                    hr = plsc.load_gather(bhr, [sv, wv, bh + off])
                    hh = plsc.load_gather(bhi, [sv, wv, bh + off])
                    rr = plsc.load_gather(brr, [sv, wv, br + off])
                    ri = plsc.load_gather(bri, [sv, wv, br + off])
                    tr = plsc.load_gather(btr, [sv, wv, bt + off])
                    ti = plsc.load_gather(bti, [sv, wv, bt + off])
                    acc = acc + (hr * rr - hh * ri) * tr \
                              + (hr * ri + hh * rr) * ti
                s = jnp.sum(acc)
                plsc.store_scatter(outv, [posv], jnp.broadcast_to(s, (L,)),
                                   mask=m0)

        start_gathers(0, 0)
        for g in range(nwin):
            if g + 1 < nwin:
                start_gathers(g + 1, (g + 1) % 2)
            drain(g % 2)
            compute(g, g % 2)

        pltpu.sync_copy(outv, out_hbm.at[pl.ds(base, b_per_w)])

    return kge_kernel(hq, hp, rq, rp, tq, tp, ere2, eim2, rre2, rim2)
